# baseline (speedup 1.0000x reference)
import functools

import jax
import jax.numpy as jnp
from jax import lax
from jax.experimental import pallas as pl
from jax.experimental.pallas import tpu as pltpu
from jax.experimental.pallas import tpu_sc as plsc

N = 10000
IN_DIM = 1433
HID = 512
OUT = 256
BM = 1000

NC, NS, NW = 2, 16, 32
ROWS = 320
CAP = 8192
CH = 128
D = 256
ACC_R = 328


def _hist(dst, n):
    return jax.ops.segment_sum(jnp.ones(dst.shape, jnp.float32), dst, num_segments=n)


def _bucket(src, dst):
    owner = dst // ROWS
    order = jnp.argsort(owner, stable=True)
    osrt = owner[order]
    counts = jnp.bincount(owner, length=NW)
    starts = jnp.cumsum(counts) - counts
    rank = jnp.arange(src.shape[0], dtype=jnp.int32) - starts[osrt]
    src_list = jnp.zeros((NW, CAP), jnp.int32).at[osrt, rank].set(src[order])
    ldst_list = jnp.full((NW, CAP), ROWS, jnp.int32).at[osrt, rank].set(
        dst[order] - osrt * ROWS)
    nch = (counts + CH - 1) // CH
    return src_list, ldst_list, nch.astype(jnp.int32).reshape(NC, NS)


def _sc_segsum(table, src_list, ldst_list, nch):
    mesh = plsc.VectorSubcoreMesh(core_axis_name="c", subcore_axis_name="s")

    @functools.partial(
        pl.kernel,
        mesh=mesh,
        out_type=jax.ShapeDtypeStruct((NW * ROWS, D), jnp.float32),
        scratch_types=[
            pltpu.VMEM((CH,), jnp.int32),
            pltpu.VMEM((CH,), jnp.int32),
            pltpu.VMEM((CH, D), jnp.float32),
            pltpu.VMEM((ACC_R, D), jnp.float32),
            pltpu.VMEM((NC, NS), jnp.int32),
            pltpu.SemaphoreType.DMA,
        ],
    )
    def k(table_h, src_h, ldst_h, nch_h, out_h, src_v, ldst_v, rows_v, acc_v, nch_v, sem):
        c = lax.axis_index("c")
        s = lax.axis_index("s")
        wid = c * NS + s

        pltpu.sync_copy(nch_h, nch_v)
        row = nch_v[c, :]
        n = jnp.int32(0)
        for kk in range(NS):
            n = jnp.where(s == kk, row[kk], n)

        def zbody(r, carry):
            for g in range(D // 16):
                acc_v[r, pl.ds(g * 16, 16)] = jnp.zeros((16,), jnp.float32)
            return carry

        lax.fori_loop(0, ACC_R, zbody, 0)

        def body(i, carry):
            off = i * CH
            pltpu.sync_copy(src_h.at[wid, pl.ds(off, CH)], src_v)
            pltpu.sync_copy(ldst_h.at[wid, pl.ds(off, CH)], ldst_v)
            pltpu.async_copy(table_h.at[src_v], rows_v, sem).wait()

            def grp(g, carry2):
                idxv = ldst_v[pl.ds(g * 16, 16)]
                for j in range(16):
                    ld = idxv[j]
                    e = g * 16 + j
                    for kq in range(D // 16):
                        sl = pl.ds(kq * 16, 16)
                        acc_v[ld, sl] = acc_v[ld, sl] + rows_v[e, sl]
                return carry2

            lax.fori_loop(0, CH // 16, grp, 0)
            return carry

        lax.fori_loop(0, n, body, 0)
        pltpu.sync_copy(acc_v.at[pl.ds(0, ROWS)], out_h.at[pl.ds(wid * ROWS, ROWS)])

    return k(table, src_list, ldst_list, nch)


def _m1_body(x_ref, w_ref, rss_ref, rsh_ref, a1_ref, b1a_ref, b1b_ref,
             g1a_ref, g1b_ref, h1a_ref, h1b_ref):
    acc = jnp.dot(x_ref[...], w_ref[...], preferred_element_type=jnp.float32)
    a1_ref[...] = acc[:, 0:HID]
    b1a_ref[...] = acc[:, HID:HID + OUT]
    b1b_ref[...] = acc[:, HID + OUT:2 * HID]
    g = acc[:, 2 * HID:3 * HID] * rss_ref[...]
    g1a_ref[...] = g[:, 0:OUT]
    g1b_ref[...] = g[:, OUT:HID]
    h = acc[:, 3 * HID:4 * HID] * rsh_ref[...]
    h1a_ref[...] = h[:, 0:OUT]
    h1b_ref[...] = h[:, OUT:HID]


def _m1(x, wcat, rs_s, rs_h):
    grid = (N // BM,)
    oh = jax.ShapeDtypeStruct((N, HID), jnp.float32)
    oq = jax.ShapeDtypeStruct((N, OUT), jnp.float32)
    return pl.pallas_call(
        _m1_body,
        grid=grid,
        in_specs=[
            pl.BlockSpec((BM, IN_DIM), lambda i: (i, 0)),
            pl.BlockSpec((IN_DIM, 4 * HID), lambda i: (0, 0)),
            pl.BlockSpec((BM, 1), lambda i: (i, 0)),
            pl.BlockSpec((BM, 1), lambda i: (i, 0)),
        ],
        out_specs=[pl.BlockSpec((BM, HID), lambda i: (i, 0))]
        + [pl.BlockSpec((BM, OUT), lambda i: (i, 0))] * 6,
        out_shape=[oh, oq, oq, oq, oq, oq, oq],
    )(x, wcat, rs_s, rs_h)


def _l2e_body(a1_ref, s1a_ref, s1b_ref, ic_ref, b1_ref, w_ref, a2_ref, b2_ref):
    s1 = jnp.concatenate([s1a_ref[...], s1b_ref[...]], axis=1)
    h = jax.nn.relu(a1_ref[...] + s1 * ic_ref[...] + b1_ref[...])
    acc = jnp.dot(h, w_ref[...], preferred_element_type=jnp.float32)
    a2_ref[...] = acc[:, 0:OUT]
    b2_ref[...] = acc[:, OUT:2 * OUT]


def _l2_euc(a1, s1a, s1b, invcnt, b1, w2cat):
    o = jax.ShapeDtypeStruct((N, OUT), jnp.float32)
    return pl.pallas_call(
        _l2e_body,
        grid=(N // BM,),
        in_specs=[
            pl.BlockSpec((BM, HID), lambda i: (i, 0)),
            pl.BlockSpec((BM, OUT), lambda i: (i, 0)),
            pl.BlockSpec((BM, OUT), lambda i: (i, 0)),
            pl.BlockSpec((BM, 1), lambda i: (i, 0)),
            pl.BlockSpec((1, HID), lambda i: (0, 0)),
            pl.BlockSpec((HID, 2 * OUT), lambda i: (0, 0)),
        ],
        out_specs=[pl.BlockSpec((BM, OUT), lambda i: (i, 0))] * 2,
        out_shape=[o, o],
    )(a1, s1a, s1b, invcnt, b1.reshape(1, HID), w2cat)


def _l2g_body(s1a_ref, s1b_ref, g1a_ref, g1b_ref, rs_ref, bg_ref, w_ref, g2_ref):
    rs = rs_ref[...]
    s1 = jnp.concatenate([s1a_ref[...], s1b_ref[...]], axis=1)
    g1 = jnp.concatenate([g1a_ref[...], g1b_ref[...]], axis=1)
    h = jax.nn.relu((s1 + g1) * rs + bg_ref[...])
    g2_ref[...] = jnp.dot(h, w_ref[...], preferred_element_type=jnp.float32) * rs


def _l2_gcn(s1a, s1b, g1a, g1b, rs, bg1, wg2):
    return pl.pallas_call(
        _l2g_body,
        grid=(N // BM,),
        in_specs=[
            pl.BlockSpec((BM, OUT), lambda i: (i, 0)),
            pl.BlockSpec((BM, OUT), lambda i: (i, 0)),
            pl.BlockSpec((BM, OUT), lambda i: (i, 0)),
            pl.BlockSpec((BM, OUT), lambda i: (i, 0)),
            pl.BlockSpec((BM, 1), lambda i: (i, 0)),
            pl.BlockSpec((1, HID), lambda i: (0, 0)),
            pl.BlockSpec((HID, OUT), lambda i: (0, 0)),
        ],
        out_specs=pl.BlockSpec((BM, OUT), lambda i: (i, 0)),
        out_shape=jax.ShapeDtypeStruct((N, OUT), jnp.float32),
    )(s1a, s1b, g1a, g1b, rs, bg1.reshape(1, HID), wg2)


def _fin_body(a2_ref, s2e_ref, ic_ref, b2_ref,
              s2s_ref, g2_ref, rss_ref, bg2_ref,
              s2h_ref, h2_ref, rsh_ref, bh2_ref, out_ref):
    euc = a2_ref[...] + s2e_ref[...] * ic_ref[...] + b2_ref[...]
    sph_pre = (s2s_ref[...] + g2_ref[...]) * rss_ref[...] + bg2_ref[...]
    n1 = jnp.maximum(jnp.sqrt(jnp.sum(sph_pre * sph_pre, axis=-1, keepdims=True)), 1e-6)
    sph = sph_pre / n1
    u = (s2h_ref[...] + h2_ref[...]) * rsh_ref[...] + bh2_ref[...]
    n2 = jnp.maximum(jnp.sqrt(jnp.sum(u * u, axis=-1, keepdims=True)), 1e-6)
    hgc = jnp.tanh(n2) * u / n2
    out_ref[0, :, :] = euc
    out_ref[1, :, :] = sph
    out_ref[2, :, :] = hgc


def _final(a2, s2e, invcnt, b2, s2s, g2p, rs_s, bg2, s2h, h2p, rs_h, bh2):
    bmat = pl.BlockSpec((BM, OUT), lambda i: (i, 0))
    brow = pl.BlockSpec((BM, 1), lambda i: (i, 0))
    bb = pl.BlockSpec((1, OUT), lambda i: (0, 0))
    return pl.pallas_call(
        _fin_body,
        grid=(N // BM,),
        in_specs=[bmat, bmat, brow, bb, bmat, bmat, brow, bb, bmat, bmat, brow, bb],
        out_specs=pl.BlockSpec((3, BM, OUT), lambda i: (0, i, 0)),
        out_shape=jax.ShapeDtypeStruct((3, N, OUT), jnp.float32),
    )(a2, s2e, invcnt, b2.reshape(1, OUT), s2s, g2p, rs_s, bg2.reshape(1, OUT),
      s2h, h2p, rs_h, bh2.reshape(1, OUT))


def kernel(node_features, euc_edge_index, sph_edge_index, hgc_edge_index, target_node_idx,
           W1_self, W1_nei, b1, W2_self, W2_nei, b2,
           Wg1, bg1, Wg2, bg2, Wh1, bh1, Wh2, bh2):
    es, ed = euc_edge_index[0], euc_edge_index[1]
    ss, sd = sph_edge_index[0], sph_edge_index[1]
    hs, hd = hgc_edge_index[0], hgc_edge_index[1]

    cnt_e = _hist(ed, N)
    invcnt = (1.0 / jnp.maximum(cnt_e, 1.0)).reshape(N, 1)
    deg_s = _hist(sd, N) + 1.0
    deg_h = _hist(hd, N) + 1.0
    rs_s = jax.lax.rsqrt(deg_s).reshape(N, 1)
    rs_h = jax.lax.rsqrt(deg_h).reshape(N, 1)

    eb = _bucket(es, ed)
    sb = _bucket(ss, sd)
    hb = _bucket(hs, hd)

    wcat = jnp.concatenate([W1_self, W1_nei, Wg1, Wh1], axis=1)
    a1, b1a, b1b, g1a, g1b, h1a, h1b = _m1(node_features, wcat, rs_s, rs_h)

    s1ea = _sc_segsum(b1a, *eb)
    s1eb = _sc_segsum(b1b, *eb)
    s1sa = _sc_segsum(g1a, *sb)
    s1sb = _sc_segsum(g1b, *sb)
    s1ha = _sc_segsum(h1a, *hb)
    s1hb = _sc_segsum(h1b, *hb)

    w2cat = jnp.concatenate([W2_self, W2_nei], axis=1)
    a2, b2t = _l2_euc(a1, s1ea, s1eb, invcnt, b1, w2cat)
    g2p = _l2_gcn(s1sa, s1sb, g1a, g1b, rs_s, bg1, Wg2)
    h2p = _l2_gcn(s1ha, s1hb, h1a, h1b, rs_h, bh1, Wh2)

    s2e = _sc_segsum(b2t, *eb)
    s2s = _sc_segsum(g2p, *sb)
    s2h = _sc_segsum(h2p, *hb)

    return _final(a2, s2e, invcnt, b2, s2s, g2p, rs_s, bg2, s2h, h2p, rs_h, bh2)


# speedup vs baseline: 1.1448x; 1.1448x over previous
import functools

import jax
import jax.numpy as jnp
from jax import lax
from jax.experimental import pallas as pl
from jax.experimental.pallas import tpu as pltpu
from jax.experimental.pallas import tpu_sc as plsc

N = 10000
IN_DIM = 1433
HID = 512
OUT = 256
BM = 1000

NC, NS, NW = 2, 16, 32
ROWS = 320
CAP = 8192
CH = 64
D = 256
ACC_R = 328


def _hist(dst, n):
    return jax.ops.segment_sum(jnp.ones(dst.shape, jnp.float32), dst, num_segments=n)


def _bucket(src, dst):
    owner = dst // ROWS
    order = jnp.argsort(owner, stable=True)
    osrt = owner[order]
    counts = jnp.bincount(owner, length=NW)
    starts = jnp.cumsum(counts) - counts
    rank = jnp.arange(src.shape[0], dtype=jnp.int32) - starts[osrt]
    src_list = jnp.zeros((NW, CAP), jnp.int32).at[osrt, rank].set(src[order])
    ldst_list = jnp.full((NW, CAP), ROWS, jnp.int32).at[osrt, rank].set(
        dst[order] - osrt * ROWS)
    pair = jnp.stack([src_list.reshape(NW, CAP // CH, CH),
                      ldst_list.reshape(NW, CAP // CH, CH)], axis=2)
    nch = (counts + CH - 1) // CH
    return pair, nch.astype(jnp.int32).reshape(NC, NS)


def _sc_segsum(table, pair, nch):
    mesh = plsc.VectorSubcoreMesh(core_axis_name="c", subcore_axis_name="s")

    @functools.partial(
        pl.kernel,
        mesh=mesh,
        out_type=jax.ShapeDtypeStruct((NW * ROWS, D), jnp.float32),
        scratch_types=[
            pltpu.VMEM((2, CH), jnp.int32),
            pltpu.VMEM((2, CH), jnp.int32),
            pltpu.VMEM((CH, D), jnp.float32),
            pltpu.VMEM((CH, D), jnp.float32),
            pltpu.VMEM((ACC_R, D), jnp.float32),
            pltpu.VMEM((NC, NS), jnp.int32),
            pltpu.SemaphoreType.DMA,
            pltpu.SemaphoreType.DMA,
        ],
    )
    def k(table_h, pair_h, nch_h, out_h, pair0, pair1, rows0, rows1, acc_v,
          nch_v, sem0, sem1):
        c = lax.axis_index("c")
        s = lax.axis_index("s")
        wid = c * NS + s

        pltpu.sync_copy(nch_h, nch_v)
        row = nch_v[c, :]
        n = jnp.int32(0)
        for kk in range(NS):
            n = jnp.where(s == kk, row[kk], n)

        def zbody(r, carry):
            for g in range(D // 16):
                acc_v[r, pl.ds(g * 16, 16)] = jnp.zeros((16,), jnp.float32)
            return carry

        lax.fori_loop(0, ACC_R, zbody, 0)

        @pl.when(n > 0)
        def _prime0():
            pltpu.sync_copy(pair_h.at[wid, 0], pair0)
            pltpu.async_copy(table_h.at[pair0.at[0]], rows0, sem0)

        @pl.when(n > 1)
        def _prime1():
            pltpu.sync_copy(pair_h.at[wid, 1], pair1)
            pltpu.async_copy(table_h.at[pair1.at[0]], rows1, sem1)

        def _accum(pair_v, rows_v):
            def grp(g, carry2):
                idxv = pair_v[1, pl.ds(g * 16, 16)]
                for j in range(16):
                    ld = idxv[j]
                    e = g * 16 + j
                    for kq in range(D // 16):
                        sl = pl.ds(kq * 16, 16)
                        plsc.addupdate(acc_v.at[ld, sl], rows_v[e, sl])
                return carry2

            lax.fori_loop(0, CH // 16, grp, 0)

        def body(i2, carry):
            for b, (pair_v, rows_v, sem) in enumerate(
                    ((pair0, rows0, sem0), (pair1, rows1, sem1))):
                cidx = i2 * 2 + b

                @pl.when(cidx < n)
                def _step(cidx=cidx, pair_v=pair_v, rows_v=rows_v, sem=sem):
                    pltpu.make_async_copy(table_h.at[pair_v.at[0]], rows_v, sem).wait()
                    _accum(pair_v, rows_v)

                    @pl.when(cidx + 2 < n)
                    def _refill():
                        pltpu.sync_copy(pair_h.at[wid, cidx + 2], pair_v)
                        pltpu.async_copy(table_h.at[pair_v.at[0]], rows_v, sem)

            return carry

        lax.fori_loop(0, (n + 1) // 2, body, 0)
        pltpu.sync_copy(acc_v.at[pl.ds(0, ROWS)], out_h.at[pl.ds(wid * ROWS, ROWS)])

    return k(table, pair, nch)


def _m1_body(x_ref, w_ref, rss_ref, rsh_ref, a1_ref, b1a_ref, b1b_ref,
             g1a_ref, g1b_ref, h1a_ref, h1b_ref):
    acc = jnp.dot(x_ref[...], w_ref[...], preferred_element_type=jnp.float32)
    a1_ref[...] = acc[:, 0:HID]
    b1a_ref[...] = acc[:, HID:HID + OUT]
    b1b_ref[...] = acc[:, HID + OUT:2 * HID]
    g = acc[:, 2 * HID:3 * HID] * rss_ref[...]
    g1a_ref[...] = g[:, 0:OUT]
    g1b_ref[...] = g[:, OUT:HID]
    h = acc[:, 3 * HID:4 * HID] * rsh_ref[...]
    h1a_ref[...] = h[:, 0:OUT]
    h1b_ref[...] = h[:, OUT:HID]


def _m1(x, wcat, rs_s, rs_h):
    grid = (N // BM,)
    oh = jax.ShapeDtypeStruct((N, HID), jnp.float32)
    oq = jax.ShapeDtypeStruct((N, OUT), jnp.float32)
    return pl.pallas_call(
        _m1_body,
        grid=grid,
        in_specs=[
            pl.BlockSpec((BM, IN_DIM), lambda i: (i, 0)),
            pl.BlockSpec((IN_DIM, 4 * HID), lambda i: (0, 0)),
            pl.BlockSpec((BM, 1), lambda i: (i, 0)),
            pl.BlockSpec((BM, 1), lambda i: (i, 0)),
        ],
        out_specs=[pl.BlockSpec((BM, HID), lambda i: (i, 0))]
        + [pl.BlockSpec((BM, OUT), lambda i: (i, 0))] * 6,
        out_shape=[oh, oq, oq, oq, oq, oq, oq],
    )(x, wcat, rs_s, rs_h)


def _l2e_body(a1_ref, s1a_ref, s1b_ref, ic_ref, b1_ref, w_ref, a2_ref, b2_ref):
    s1 = jnp.concatenate([s1a_ref[...], s1b_ref[...]], axis=1)
    h = jax.nn.relu(a1_ref[...] + s1 * ic_ref[...] + b1_ref[...])
    acc = jnp.dot(h, w_ref[...], preferred_element_type=jnp.float32)
    a2_ref[...] = acc[:, 0:OUT]
    b2_ref[...] = acc[:, OUT:2 * OUT]


def _l2_euc(a1, s1a, s1b, invcnt, b1, w2cat):
    o = jax.ShapeDtypeStruct((N, OUT), jnp.float32)
    return pl.pallas_call(
        _l2e_body,
        grid=(N // BM,),
        in_specs=[
            pl.BlockSpec((BM, HID), lambda i: (i, 0)),
            pl.BlockSpec((BM, OUT), lambda i: (i, 0)),
            pl.BlockSpec((BM, OUT), lambda i: (i, 0)),
            pl.BlockSpec((BM, 1), lambda i: (i, 0)),
            pl.BlockSpec((1, HID), lambda i: (0, 0)),
            pl.BlockSpec((HID, 2 * OUT), lambda i: (0, 0)),
        ],
        out_specs=[pl.BlockSpec((BM, OUT), lambda i: (i, 0))] * 2,
        out_shape=[o, o],
    )(a1, s1a, s1b, invcnt, b1.reshape(1, HID), w2cat)


def _l2g_body(s1a_ref, s1b_ref, g1a_ref, g1b_ref, rs_ref, bg_ref, w_ref, g2_ref):
    rs = rs_ref[...]
    s1 = jnp.concatenate([s1a_ref[...], s1b_ref[...]], axis=1)
    g1 = jnp.concatenate([g1a_ref[...], g1b_ref[...]], axis=1)
    h = jax.nn.relu((s1 + g1) * rs + bg_ref[...])
    g2_ref[...] = jnp.dot(h, w_ref[...], preferred_element_type=jnp.float32) * rs


def _l2_gcn(s1a, s1b, g1a, g1b, rs, bg1, wg2):
    return pl.pallas_call(
        _l2g_body,
        grid=(N // BM,),
        in_specs=[
            pl.BlockSpec((BM, OUT), lambda i: (i, 0)),
            pl.BlockSpec((BM, OUT), lambda i: (i, 0)),
            pl.BlockSpec((BM, OUT), lambda i: (i, 0)),
            pl.BlockSpec((BM, OUT), lambda i: (i, 0)),
            pl.BlockSpec((BM, 1), lambda i: (i, 0)),
            pl.BlockSpec((1, HID), lambda i: (0, 0)),
            pl.BlockSpec((HID, OUT), lambda i: (0, 0)),
        ],
        out_specs=pl.BlockSpec((BM, OUT), lambda i: (i, 0)),
        out_shape=jax.ShapeDtypeStruct((N, OUT), jnp.float32),
    )(s1a, s1b, g1a, g1b, rs, bg1.reshape(1, HID), wg2)


def _fin_body(a2_ref, s2e_ref, ic_ref, b2_ref,
              s2s_ref, g2_ref, rss_ref, bg2_ref,
              s2h_ref, h2_ref, rsh_ref, bh2_ref, out_ref):
    euc = a2_ref[...] + s2e_ref[...] * ic_ref[...] + b2_ref[...]
    sph_pre = (s2s_ref[...] + g2_ref[...]) * rss_ref[...] + bg2_ref[...]
    n1 = jnp.maximum(jnp.sqrt(jnp.sum(sph_pre * sph_pre, axis=-1, keepdims=True)), 1e-6)
    sph = sph_pre / n1
    u = (s2h_ref[...] + h2_ref[...]) * rsh_ref[...] + bh2_ref[...]
    n2 = jnp.maximum(jnp.sqrt(jnp.sum(u * u, axis=-1, keepdims=True)), 1e-6)
    hgc = jnp.tanh(n2) * u / n2
    out_ref[0, :, :] = euc
    out_ref[1, :, :] = sph
    out_ref[2, :, :] = hgc


def _final(a2, s2e, invcnt, b2, s2s, g2p, rs_s, bg2, s2h, h2p, rs_h, bh2):
    bmat = pl.BlockSpec((BM, OUT), lambda i: (i, 0))
    brow = pl.BlockSpec((BM, 1), lambda i: (i, 0))
    bb = pl.BlockSpec((1, OUT), lambda i: (0, 0))
    return pl.pallas_call(
        _fin_body,
        grid=(N // BM,),
        in_specs=[bmat, bmat, brow, bb, bmat, bmat, brow, bb, bmat, bmat, brow, bb],
        out_specs=pl.BlockSpec((3, BM, OUT), lambda i: (0, i, 0)),
        out_shape=jax.ShapeDtypeStruct((3, N, OUT), jnp.float32),
    )(a2, s2e, invcnt, b2.reshape(1, OUT), s2s, g2p, rs_s, bg2.reshape(1, OUT),
      s2h, h2p, rs_h, bh2.reshape(1, OUT))


def kernel(node_features, euc_edge_index, sph_edge_index, hgc_edge_index, target_node_idx,
           W1_self, W1_nei, b1, W2_self, W2_nei, b2,
           Wg1, bg1, Wg2, bg2, Wh1, bh1, Wh2, bh2):
    es, ed = euc_edge_index[0], euc_edge_index[1]
    ss, sd = sph_edge_index[0], sph_edge_index[1]
    hs, hd = hgc_edge_index[0], hgc_edge_index[1]

    cnt_e = _hist(ed, N)
    invcnt = (1.0 / jnp.maximum(cnt_e, 1.0)).reshape(N, 1)
    deg_s = _hist(sd, N) + 1.0
    deg_h = _hist(hd, N) + 1.0
    rs_s = jax.lax.rsqrt(deg_s).reshape(N, 1)
    rs_h = jax.lax.rsqrt(deg_h).reshape(N, 1)

    eb = _bucket(es, ed)
    sb = _bucket(ss, sd)
    hb = _bucket(hs, hd)

    wcat = jnp.concatenate([W1_self, W1_nei, Wg1, Wh1], axis=1)
    a1, b1a, b1b, g1a, g1b, h1a, h1b = _m1(node_features, wcat, rs_s, rs_h)

    s1ea = _sc_segsum(b1a, *eb)
    s1eb = _sc_segsum(b1b, *eb)
    s1sa = _sc_segsum(g1a, *sb)
    s1sb = _sc_segsum(g1b, *sb)
    s1ha = _sc_segsum(h1a, *hb)
    s1hb = _sc_segsum(h1b, *hb)

    w2cat = jnp.concatenate([W2_self, W2_nei], axis=1)
    a2, b2t = _l2_euc(a1, s1ea, s1eb, invcnt, b1, w2cat)
    g2p = _l2_gcn(s1sa, s1sb, g1a, g1b, rs_s, bg1, Wg2)
    h2p = _l2_gcn(s1ha, s1hb, h1a, h1b, rs_h, bh1, Wh2)

    s2e = _sc_segsum(b2t, *eb)
    s2s = _sc_segsum(g2p, *sb)
    s2h = _sc_segsum(h2p, *hb)

    return _final(a2, s2e, invcnt, b2, s2s, g2p, rs_s, bg2, s2h, h2p, rs_h, bh2)


# speedup vs baseline: 1.1778x; 1.0288x over previous
import functools

import jax
import jax.numpy as jnp
from jax import lax
from jax.experimental import pallas as pl
from jax.experimental.pallas import tpu as pltpu
from jax.experimental.pallas import tpu_sc as plsc

N = 10000
IN_DIM = 1433
HID = 512
OUT = 256
BM = 1000

NC, NS, NW = 2, 16, 32
ROWS = 320
CAP = 8192
CH = 64
PF = 32
D = 256
ACC_R = 328


def _hist(dst, n):
    return jax.ops.segment_sum(jnp.ones(dst.shape, jnp.float32), dst, num_segments=n)


def _bucket(src, dst):
    owner = dst // ROWS
    order = jnp.argsort(owner, stable=True)
    osrt = owner[order]
    counts = jnp.bincount(owner, length=NW)
    starts = jnp.cumsum(counts) - counts
    rank = jnp.arange(src.shape[0], dtype=jnp.int32) - starts[osrt]
    src_list = jnp.zeros((NW, CAP), jnp.int32).at[osrt, rank].set(src[order])
    ldst_list = jnp.full((NW, CAP), ROWS, jnp.int32).at[osrt, rank].set(
        dst[order] - osrt * ROWS)
    pair = jnp.concatenate([src_list.reshape(NW, CAP // CH, CH),
                            ldst_list.reshape(NW, CAP // CH, CH)], axis=2)
    nch = (counts + CH - 1) // CH
    return pair, nch.astype(jnp.int32).reshape(NC, NS)


def _sc_segsum(table, pair, nch):
    mesh = plsc.VectorSubcoreMesh(core_axis_name="c", subcore_axis_name="s")

    @functools.partial(
        pl.kernel,
        mesh=mesh,
        out_type=jax.ShapeDtypeStruct((NW * ROWS, D), jnp.float32),
        scratch_types=[
            pltpu.VMEM((2, PF, 2 * CH), jnp.int32),
            pltpu.VMEM((CH, D), jnp.float32),
            pltpu.VMEM((CH, D), jnp.float32),
            pltpu.VMEM((ACC_R, D), jnp.float32),
            pltpu.VMEM((NC, NS), jnp.int32),
            pltpu.SemaphoreType.DMA,
            pltpu.SemaphoreType.DMA,
        ],
    )
    def k(table_h, pair_h, nch_h, out_h, pair_big, rows0, rows1, acc_v,
          nch_v, sem0, sem1):
        c = lax.axis_index("c")
        s = lax.axis_index("s")
        wid = c * NS + s

        pltpu.sync_copy(nch_h, nch_v)
        row = nch_v[c, :]
        n = jnp.int32(0)
        for kk in range(NS):
            n = jnp.where(s == kk, row[kk], n)

        def zbody(r, carry):
            for g in range(D // 16):
                acc_v[r, pl.ds(g * 16, 16)] = jnp.zeros((16,), jnp.float32)
            return carry

        lax.fori_loop(0, ACC_R, zbody, 0)

        @pl.when(n > 0)
        def _prime0():
            pltpu.sync_copy(pair_h.at[wid, pl.ds(0, PF)], pair_big.at[0])
            pltpu.async_copy(table_h.at[pair_big.at[0, 0, pl.ds(0, CH)]], rows0, sem0)

        @pl.when(n > 1)
        def _prime1():
            pltpu.async_copy(table_h.at[pair_big.at[0, 1, pl.ds(0, CH)]], rows1, sem1)

        def _accum(cidx, rows_v):
            hb = (cidx // PF) & 1
            sl_c = cidx & (PF - 1)

            def grp(g, carry2):
                idxv = pair_big[hb, sl_c, pl.ds(CH + g * 16, 16)]
                for j in range(16):
                    ld = idxv[j]
                    e = g * 16 + j
                    for kq in range(D // 16):
                        sl = pl.ds(kq * 16, 16)
                        plsc.addupdate(acc_v.at[ld, sl], rows_v[e, sl])
                return carry2

            lax.fori_loop(0, CH // 16, grp, 0)

        def body(i2, carry):
            for b, (rows_v, sem) in enumerate(((rows0, sem0), (rows1, sem1))):
                cidx = i2 * 2 + b

                @pl.when(cidx < n)
                def _step(cidx=cidx, rows_v=rows_v, sem=sem):
                    pltpu.make_async_copy(
                        table_h.at[pair_big.at[0, 0, pl.ds(0, CH)]], rows_v, sem).wait()
                    _accum(cidx, rows_v)
                    q = cidx + 2

                    @pl.when(q < n)
                    def _refill():
                        qh = (q // PF) & 1
                        qs = q & (PF - 1)

                        @pl.when(qs == 0)
                        def _page():
                            q_al = pl.multiple_of(q, PF)
                            pltpu.sync_copy(pair_h.at[wid, pl.ds(q_al, PF)],
                                            pair_big.at[qh])

                        pltpu.async_copy(
                            table_h.at[pair_big.at[qh, qs, pl.ds(0, CH)]],
                            rows_v, sem)

            return carry

        lax.fori_loop(0, (n + 1) // 2, body, 0)
        pltpu.sync_copy(acc_v.at[pl.ds(0, ROWS)], out_h.at[pl.ds(wid * ROWS, ROWS)])

    return k(table, pair, nch)


def _m1_body(x_ref, w_ref, rss_ref, rsh_ref, a1_ref, b1a_ref, b1b_ref,
             g1a_ref, g1b_ref, h1a_ref, h1b_ref):
    acc = jnp.dot(x_ref[...], w_ref[...], preferred_element_type=jnp.float32)
    a1_ref[...] = acc[:, 0:HID]
    b1a_ref[...] = acc[:, HID:HID + OUT]
    b1b_ref[...] = acc[:, HID + OUT:2 * HID]
    g = acc[:, 2 * HID:3 * HID] * rss_ref[...]
    g1a_ref[...] = g[:, 0:OUT]
    g1b_ref[...] = g[:, OUT:HID]
    h = acc[:, 3 * HID:4 * HID] * rsh_ref[...]
    h1a_ref[...] = h[:, 0:OUT]
    h1b_ref[...] = h[:, OUT:HID]


def _m1(x, wcat, rs_s, rs_h):
    grid = (N // BM,)
    oh = jax.ShapeDtypeStruct((N, HID), jnp.float32)
    oq = jax.ShapeDtypeStruct((N, OUT), jnp.float32)
    return pl.pallas_call(
        _m1_body,
        grid=grid,
        in_specs=[
            pl.BlockSpec((BM, IN_DIM), lambda i: (i, 0)),
            pl.BlockSpec((IN_DIM, 4 * HID), lambda i: (0, 0)),
            pl.BlockSpec((BM, 1), lambda i: (i, 0)),
            pl.BlockSpec((BM, 1), lambda i: (i, 0)),
        ],
        out_specs=[pl.BlockSpec((BM, HID), lambda i: (i, 0))]
        + [pl.BlockSpec((BM, OUT), lambda i: (i, 0))] * 6,
        out_shape=[oh, oq, oq, oq, oq, oq, oq],
    )(x, wcat, rs_s, rs_h)


def _l2e_body(a1_ref, s1a_ref, s1b_ref, ic_ref, b1_ref, w_ref, a2_ref, b2_ref):
    s1 = jnp.concatenate([s1a_ref[...], s1b_ref[...]], axis=1)
    h = jax.nn.relu(a1_ref[...] + s1 * ic_ref[...] + b1_ref[...])
    acc = jnp.dot(h, w_ref[...], preferred_element_type=jnp.float32)
    a2_ref[...] = acc[:, 0:OUT]
    b2_ref[...] = acc[:, OUT:2 * OUT]


def _l2_euc(a1, s1a, s1b, invcnt, b1, w2cat):
    o = jax.ShapeDtypeStruct((N, OUT), jnp.float32)
    return pl.pallas_call(
        _l2e_body,
        grid=(N // BM,),
        in_specs=[
            pl.BlockSpec((BM, HID), lambda i: (i, 0)),
            pl.BlockSpec((BM, OUT), lambda i: (i, 0)),
            pl.BlockSpec((BM, OUT), lambda i: (i, 0)),
            pl.BlockSpec((BM, 1), lambda i: (i, 0)),
            pl.BlockSpec((1, HID), lambda i: (0, 0)),
            pl.BlockSpec((HID, 2 * OUT), lambda i: (0, 0)),
        ],
        out_specs=[pl.BlockSpec((BM, OUT), lambda i: (i, 0))] * 2,
        out_shape=[o, o],
    )(a1, s1a, s1b, invcnt, b1.reshape(1, HID), w2cat)


def _l2g_body(s1a_ref, s1b_ref, g1a_ref, g1b_ref, rs_ref, bg_ref, w_ref, g2_ref):
    rs = rs_ref[...]
    s1 = jnp.concatenate([s1a_ref[...], s1b_ref[...]], axis=1)
    g1 = jnp.concatenate([g1a_ref[...], g1b_ref[...]], axis=1)
    h = jax.nn.relu((s1 + g1) * rs + bg_ref[...])
    g2_ref[...] = jnp.dot(h, w_ref[...], preferred_element_type=jnp.float32) * rs


def _l2_gcn(s1a, s1b, g1a, g1b, rs, bg1, wg2):
    return pl.pallas_call(
        _l2g_body,
        grid=(N // BM,),
        in_specs=[
            pl.BlockSpec((BM, OUT), lambda i: (i, 0)),
            pl.BlockSpec((BM, OUT), lambda i: (i, 0)),
            pl.BlockSpec((BM, OUT), lambda i: (i, 0)),
            pl.BlockSpec((BM, OUT), lambda i: (i, 0)),
            pl.BlockSpec((BM, 1), lambda i: (i, 0)),
            pl.BlockSpec((1, HID), lambda i: (0, 0)),
            pl.BlockSpec((HID, OUT), lambda i: (0, 0)),
        ],
        out_specs=pl.BlockSpec((BM, OUT), lambda i: (i, 0)),
        out_shape=jax.ShapeDtypeStruct((N, OUT), jnp.float32),
    )(s1a, s1b, g1a, g1b, rs, bg1.reshape(1, HID), wg2)


def _fin_body(a2_ref, s2e_ref, ic_ref, b2_ref,
              s2s_ref, g2_ref, rss_ref, bg2_ref,
              s2h_ref, h2_ref, rsh_ref, bh2_ref, out_ref):
    euc = a2_ref[...] + s2e_ref[...] * ic_ref[...] + b2_ref[...]
    sph_pre = (s2s_ref[...] + g2_ref[...]) * rss_ref[...] + bg2_ref[...]
    n1 = jnp.maximum(jnp.sqrt(jnp.sum(sph_pre * sph_pre, axis=-1, keepdims=True)), 1e-6)
    sph = sph_pre / n1
    u = (s2h_ref[...] + h2_ref[...]) * rsh_ref[...] + bh2_ref[...]
    n2 = jnp.maximum(jnp.sqrt(jnp.sum(u * u, axis=-1, keepdims=True)), 1e-6)
    hgc = jnp.tanh(n2) * u / n2
    out_ref[0, :, :] = euc
    out_ref[1, :, :] = sph
    out_ref[2, :, :] = hgc


def _final(a2, s2e, invcnt, b2, s2s, g2p, rs_s, bg2, s2h, h2p, rs_h, bh2):
    bmat = pl.BlockSpec((BM, OUT), lambda i: (i, 0))
    brow = pl.BlockSpec((BM, 1), lambda i: (i, 0))
    bb = pl.BlockSpec((1, OUT), lambda i: (0, 0))
    return pl.pallas_call(
        _fin_body,
        grid=(N // BM,),
        in_specs=[bmat, bmat, brow, bb, bmat, bmat, brow, bb, bmat, bmat, brow, bb],
        out_specs=pl.BlockSpec((3, BM, OUT), lambda i: (0, i, 0)),
        out_shape=jax.ShapeDtypeStruct((3, N, OUT), jnp.float32),
    )(a2, s2e, invcnt, b2.reshape(1, OUT), s2s, g2p, rs_s, bg2.reshape(1, OUT),
      s2h, h2p, rs_h, bh2.reshape(1, OUT))


def kernel(node_features, euc_edge_index, sph_edge_index, hgc_edge_index, target_node_idx,
           W1_self, W1_nei, b1, W2_self, W2_nei, b2,
           Wg1, bg1, Wg2, bg2, Wh1, bh1, Wh2, bh2):
    es, ed = euc_edge_index[0], euc_edge_index[1]
    ss, sd = sph_edge_index[0], sph_edge_index[1]
    hs, hd = hgc_edge_index[0], hgc_edge_index[1]

    cnt_e = _hist(ed, N)
    invcnt = (1.0 / jnp.maximum(cnt_e, 1.0)).reshape(N, 1)
    deg_s = _hist(sd, N) + 1.0
    deg_h = _hist(hd, N) + 1.0
    rs_s = jax.lax.rsqrt(deg_s).reshape(N, 1)
    rs_h = jax.lax.rsqrt(deg_h).reshape(N, 1)

    eb = _bucket(es, ed)
    sb = _bucket(ss, sd)
    hb = _bucket(hs, hd)

    wcat = jnp.concatenate([W1_self, W1_nei, Wg1, Wh1], axis=1)
    a1, b1a, b1b, g1a, g1b, h1a, h1b = _m1(node_features, wcat, rs_s, rs_h)

    s1ea = _sc_segsum(b1a, *eb)
    s1eb = _sc_segsum(b1b, *eb)
    s1sa = _sc_segsum(g1a, *sb)
    s1sb = _sc_segsum(g1b, *sb)
    s1ha = _sc_segsum(h1a, *hb)
    s1hb = _sc_segsum(h1b, *hb)

    w2cat = jnp.concatenate([W2_self, W2_nei], axis=1)
    a2, b2t = _l2_euc(a1, s1ea, s1eb, invcnt, b1, w2cat)
    g2p = _l2_gcn(s1sa, s1sb, g1a, g1b, rs_s, bg1, Wg2)
    h2p = _l2_gcn(s1ha, s1hb, h1a, h1b, rs_h, bh1, Wh2)

    s2e = _sc_segsum(b2t, *eb)
    s2s = _sc_segsum(g2p, *sb)
    s2h = _sc_segsum(h2p, *hb)

    return _final(a2, s2e, invcnt, b2, s2s, g2p, rs_s, bg2, s2h, h2p, rs_h, bh2)


# speedup vs baseline: 1.2005x; 1.0193x over previous
import functools

import jax
import jax.numpy as jnp
from jax import lax
from jax.experimental import pallas as pl
from jax.experimental.pallas import tpu as pltpu
from jax.experimental.pallas import tpu_sc as plsc

N = 10000
IN_DIM = 1433
HID = 512
OUT = 256
BM = 1000

NC, NS, NW = 2, 16, 32
ROWS = 320
CAP = 8192
CH = 64
PF = 32
D = 256
ACC_R = 328


def _bucket(src, dst):
    owner = dst // ROWS
    order = jnp.argsort(owner, stable=True)
    osrt = owner[order]
    counts = jnp.bincount(owner, length=NW)
    starts = jnp.cumsum(counts) - counts
    rank = jnp.arange(src.shape[0], dtype=jnp.int32) - starts[osrt]
    src_list = jnp.zeros((NW, CAP), jnp.int32).at[osrt, rank].set(src[order])
    ldst_list = jnp.full((NW, CAP), ROWS, jnp.int32).at[osrt, rank].set(
        dst[order] - osrt * ROWS)
    pair = jnp.concatenate([src_list.reshape(NW, CAP // CH, CH),
                            ldst_list.reshape(NW, CAP // CH, CH)], axis=2)
    nch = (counts + CH - 1) // CH
    return pair, nch.astype(jnp.int32).reshape(NC, NS)


def _sc_segsum(table, pair, nch):
    mesh = plsc.VectorSubcoreMesh(core_axis_name="c", subcore_axis_name="s")

    @functools.partial(
        pl.kernel,
        mesh=mesh,
        out_type=jax.ShapeDtypeStruct((NW * ROWS, D), jnp.float32),
        scratch_types=[
            pltpu.VMEM((2, PF, 2 * CH), jnp.int32),
            pltpu.VMEM((CH, D), jnp.float32),
            pltpu.VMEM((CH, D), jnp.float32),
            pltpu.VMEM((ACC_R, D), jnp.float32),
            pltpu.VMEM((NC, NS), jnp.int32),
            pltpu.SemaphoreType.DMA,
            pltpu.SemaphoreType.DMA,
        ],
    )
    def k(table_h, pair_h, nch_h, out_h, pair_big, rows0, rows1, acc_v,
          nch_v, sem0, sem1):
        c = lax.axis_index("c")
        s = lax.axis_index("s")
        wid = c * NS + s

        pltpu.sync_copy(nch_h, nch_v)
        row = nch_v[c, :]
        n = jnp.int32(0)
        for kk in range(NS):
            n = jnp.where(s == kk, row[kk], n)

        def zbody(r, carry):
            for g in range(D // 16):
                acc_v[r, pl.ds(g * 16, 16)] = jnp.zeros((16,), jnp.float32)
            return carry

        lax.fori_loop(0, ACC_R, zbody, 0)

        @pl.when(n > 0)
        def _prime0():
            pltpu.sync_copy(pair_h.at[wid, pl.ds(0, PF)], pair_big.at[0])
            pltpu.async_copy(table_h.at[pair_big.at[0, 0, pl.ds(0, CH)]], rows0, sem0)

        @pl.when(n > 1)
        def _prime1():
            pltpu.async_copy(table_h.at[pair_big.at[0, 1, pl.ds(0, CH)]], rows1, sem1)

        def _accum(cidx, rows_v):
            hb = (cidx // PF) & 1
            sl_c = cidx & (PF - 1)

            def grp(g, carry2):
                idxv = pair_big[hb, sl_c, pl.ds(CH + g * 16, 16)]
                for j in range(16):
                    ld = idxv[j]
                    e = g * 16 + j
                    for kq in range(D // 16):
                        sl = pl.ds(kq * 16, 16)
                        plsc.addupdate(acc_v.at[ld, sl], rows_v[e, sl])
                return carry2

            lax.fori_loop(0, CH // 16, grp, 0)

        def body(i2, carry):
            for b, (rows_v, sem) in enumerate(((rows0, sem0), (rows1, sem1))):
                cidx = i2 * 2 + b

                @pl.when(cidx < n)
                def _step(cidx=cidx, rows_v=rows_v, sem=sem):
                    pltpu.make_async_copy(
                        table_h.at[pair_big.at[0, 0, pl.ds(0, CH)]], rows_v, sem).wait()
                    _accum(cidx, rows_v)
                    q = cidx + 2

                    @pl.when(q < n)
                    def _refill():
                        qh = (q // PF) & 1
                        qs = q & (PF - 1)

                        @pl.when(qs == 0)
                        def _page():
                            q_al = pl.multiple_of(q, PF)
                            pltpu.sync_copy(pair_h.at[wid, pl.ds(q_al, PF)],
                                            pair_big.at[qh])

                        pltpu.async_copy(
                            table_h.at[pair_big.at[qh, qs, pl.ds(0, CH)]],
                            rows_v, sem)

            return carry

        lax.fori_loop(0, (n + 1) // 2, body, 0)
        pltpu.sync_copy(acc_v.at[pl.ds(0, ROWS)], out_h.at[pl.ds(wid * ROWS, ROWS)])

    return k(table, pair, nch)


def _sc_count(pair, nch):
    mesh = plsc.VectorSubcoreMesh(core_axis_name="c", subcore_axis_name="s")

    @functools.partial(
        pl.kernel,
        mesh=mesh,
        out_type=jax.ShapeDtypeStruct((NW * ROWS, 16), jnp.float32),
        scratch_types=[
            pltpu.VMEM((PF, 2 * CH), jnp.int32),
            pltpu.VMEM((ACC_R, 16), jnp.float32),
            pltpu.VMEM((NC, NS), jnp.int32),
        ],
    )
    def k(pair_h, nch_h, out_h, pair_big, cnt_v, nch_v):
        c = lax.axis_index("c")
        s = lax.axis_index("s")
        wid = c * NS + s

        pltpu.sync_copy(nch_h, nch_v)
        row = nch_v[c, :]
        n = jnp.int32(0)
        for kk in range(NS):
            n = jnp.where(s == kk, row[kk], n)

        def zbody(r, carry):
            cnt_v[r, :] = jnp.zeros((16,), jnp.float32)
            return carry

        lax.fori_loop(0, ACC_R, zbody, 0)
        ones = jnp.ones((16,), jnp.float32)

        def body(cidx, carry):
            qs = cidx & (PF - 1)

            @pl.when(qs == 0)
            def _page():
                q_al = pl.multiple_of(cidx, PF)
                pltpu.sync_copy(pair_h.at[wid, pl.ds(q_al, PF)], pair_big)

            def grp(g, carry2):
                idxv = pair_big[qs, pl.ds(CH + g * 16, 16)]
                for j in range(16):
                    ld = idxv[j]
                    plsc.addupdate(cnt_v.at[ld, :], ones)
                return carry2

            lax.fori_loop(0, CH // 16, grp, 0)
            return carry

        lax.fori_loop(0, n, body, 0)
        pltpu.sync_copy(cnt_v.at[pl.ds(0, ROWS)], out_h.at[pl.ds(wid * ROWS, ROWS)])

    return k(pair, nch)


def _m1_body(x_ref, w_ref, cns_ref, cnh_ref, a1_ref, b1a_ref, b1b_ref,
             g1a_ref, g1b_ref, h1a_ref, h1b_ref):
    acc = jnp.dot(x_ref[...], w_ref[...], preferred_element_type=jnp.float32)
    rss = lax.rsqrt(cns_ref[...] + 1.0)
    rsh = lax.rsqrt(cnh_ref[...] + 1.0)
    a1_ref[...] = acc[:, 0:HID]
    b1a_ref[...] = acc[:, HID:HID + OUT]
    b1b_ref[...] = acc[:, HID + OUT:2 * HID]
    g = acc[:, 2 * HID:3 * HID] * rss
    g1a_ref[...] = g[:, 0:OUT]
    g1b_ref[...] = g[:, OUT:HID]
    h = acc[:, 3 * HID:4 * HID] * rsh
    h1a_ref[...] = h[:, 0:OUT]
    h1b_ref[...] = h[:, OUT:HID]


def _m1(x, wcat, rs_s, rs_h):
    grid = (N // BM,)
    oh = jax.ShapeDtypeStruct((N, HID), jnp.float32)
    oq = jax.ShapeDtypeStruct((N, OUT), jnp.float32)
    return pl.pallas_call(
        _m1_body,
        grid=grid,
        in_specs=[
            pl.BlockSpec((BM, IN_DIM), lambda i: (i, 0)),
            pl.BlockSpec((IN_DIM, 4 * HID), lambda i: (0, 0)),
            pl.BlockSpec((BM, 1), lambda i: (i, 0)),
            pl.BlockSpec((BM, 1), lambda i: (i, 0)),
        ],
        out_specs=[pl.BlockSpec((BM, HID), lambda i: (i, 0))]
        + [pl.BlockSpec((BM, OUT), lambda i: (i, 0))] * 6,
        out_shape=[oh, oq, oq, oq, oq, oq, oq],
    )(x, wcat, rs_s, rs_h)


def _l2e_body(a1_ref, s1a_ref, s1b_ref, cne_ref, b1_ref, w_ref, a2_ref, b2_ref):
    ic = 1.0 / jnp.maximum(cne_ref[...], 1.0)
    s1 = jnp.concatenate([s1a_ref[...], s1b_ref[...]], axis=1)
    h = jax.nn.relu(a1_ref[...] + s1 * ic + b1_ref[...])
    acc = jnp.dot(h, w_ref[...], preferred_element_type=jnp.float32)
    a2_ref[...] = acc[:, 0:OUT]
    b2_ref[...] = acc[:, OUT:2 * OUT]


def _l2_euc(a1, s1a, s1b, invcnt, b1, w2cat):
    o = jax.ShapeDtypeStruct((N, OUT), jnp.float32)
    return pl.pallas_call(
        _l2e_body,
        grid=(N // BM,),
        in_specs=[
            pl.BlockSpec((BM, HID), lambda i: (i, 0)),
            pl.BlockSpec((BM, OUT), lambda i: (i, 0)),
            pl.BlockSpec((BM, OUT), lambda i: (i, 0)),
            pl.BlockSpec((BM, 1), lambda i: (i, 0)),
            pl.BlockSpec((1, HID), lambda i: (0, 0)),
            pl.BlockSpec((HID, 2 * OUT), lambda i: (0, 0)),
        ],
        out_specs=[pl.BlockSpec((BM, OUT), lambda i: (i, 0))] * 2,
        out_shape=[o, o],
    )(a1, s1a, s1b, invcnt, b1.reshape(1, HID), w2cat)


def _l2g_body(s1a_ref, s1b_ref, g1a_ref, g1b_ref, cn_ref, bg_ref, w_ref, g2_ref):
    rs = lax.rsqrt(cn_ref[...] + 1.0)
    s1 = jnp.concatenate([s1a_ref[...], s1b_ref[...]], axis=1)
    g1 = jnp.concatenate([g1a_ref[...], g1b_ref[...]], axis=1)
    h = jax.nn.relu((s1 + g1) * rs + bg_ref[...])
    g2_ref[...] = jnp.dot(h, w_ref[...], preferred_element_type=jnp.float32) * rs


def _l2_gcn(s1a, s1b, g1a, g1b, rs, bg1, wg2):
    return pl.pallas_call(
        _l2g_body,
        grid=(N // BM,),
        in_specs=[
            pl.BlockSpec((BM, OUT), lambda i: (i, 0)),
            pl.BlockSpec((BM, OUT), lambda i: (i, 0)),
            pl.BlockSpec((BM, OUT), lambda i: (i, 0)),
            pl.BlockSpec((BM, OUT), lambda i: (i, 0)),
            pl.BlockSpec((BM, 1), lambda i: (i, 0)),
            pl.BlockSpec((1, HID), lambda i: (0, 0)),
            pl.BlockSpec((HID, OUT), lambda i: (0, 0)),
        ],
        out_specs=pl.BlockSpec((BM, OUT), lambda i: (i, 0)),
        out_shape=jax.ShapeDtypeStruct((N, OUT), jnp.float32),
    )(s1a, s1b, g1a, g1b, rs, bg1.reshape(1, HID), wg2)


def _fin_body(a2_ref, s2e_ref, cne_ref, b2_ref,
              s2s_ref, g2_ref, cns_ref, bg2_ref,
              s2h_ref, h2_ref, cnh_ref, bh2_ref, out_ref):
    ic = 1.0 / jnp.maximum(cne_ref[...], 1.0)
    euc = a2_ref[...] + s2e_ref[...] * ic + b2_ref[...]
    sph_pre = (s2s_ref[...] + g2_ref[...]) * lax.rsqrt(cns_ref[...] + 1.0) + bg2_ref[...]
    n1 = jnp.maximum(jnp.sqrt(jnp.sum(sph_pre * sph_pre, axis=-1, keepdims=True)), 1e-6)
    sph = sph_pre / n1
    u = (s2h_ref[...] + h2_ref[...]) * lax.rsqrt(cnh_ref[...] + 1.0) + bh2_ref[...]
    n2 = jnp.maximum(jnp.sqrt(jnp.sum(u * u, axis=-1, keepdims=True)), 1e-6)
    hgc = jnp.tanh(n2) * u / n2
    out_ref[0, :, :] = euc
    out_ref[1, :, :] = sph
    out_ref[2, :, :] = hgc


def _final(a2, s2e, invcnt, b2, s2s, g2p, rs_s, bg2, s2h, h2p, rs_h, bh2):
    bmat = pl.BlockSpec((BM, OUT), lambda i: (i, 0))
    brow = pl.BlockSpec((BM, 1), lambda i: (i, 0))
    bb = pl.BlockSpec((1, OUT), lambda i: (0, 0))
    return pl.pallas_call(
        _fin_body,
        grid=(N // BM,),
        in_specs=[bmat, bmat, brow, bb, bmat, bmat, brow, bb, bmat, bmat, brow, bb],
        out_specs=pl.BlockSpec((3, BM, OUT), lambda i: (0, i, 0)),
        out_shape=jax.ShapeDtypeStruct((3, N, OUT), jnp.float32),
    )(a2, s2e, invcnt, b2.reshape(1, OUT), s2s, g2p, rs_s, bg2.reshape(1, OUT),
      s2h, h2p, rs_h, bh2.reshape(1, OUT))


def kernel(node_features, euc_edge_index, sph_edge_index, hgc_edge_index, target_node_idx,
           W1_self, W1_nei, b1, W2_self, W2_nei, b2,
           Wg1, bg1, Wg2, bg2, Wh1, bh1, Wh2, bh2):
    es, ed = euc_edge_index[0], euc_edge_index[1]
    ss, sd = sph_edge_index[0], sph_edge_index[1]
    hs, hd = hgc_edge_index[0], hgc_edge_index[1]

    eb = _bucket(es, ed)
    sb = _bucket(ss, sd)
    hb = _bucket(hs, hd)

    cnt_e = _sc_count(*eb)[:N, 0:1]
    cnt_s = _sc_count(*sb)[:N, 0:1]
    cnt_h = _sc_count(*hb)[:N, 0:1]

    wcat = jnp.concatenate([W1_self, W1_nei, Wg1, Wh1], axis=1)
    a1, b1a, b1b, g1a, g1b, h1a, h1b = _m1(node_features, wcat, cnt_s, cnt_h)

    s1ea = _sc_segsum(b1a, *eb)
    s1eb = _sc_segsum(b1b, *eb)
    s1sa = _sc_segsum(g1a, *sb)
    s1sb = _sc_segsum(g1b, *sb)
    s1ha = _sc_segsum(h1a, *hb)
    s1hb = _sc_segsum(h1b, *hb)

    w2cat = jnp.concatenate([W2_self, W2_nei], axis=1)
    a2, b2t = _l2_euc(a1, s1ea, s1eb, cnt_e, b1, w2cat)
    g2p = _l2_gcn(s1sa, s1sb, g1a, g1b, cnt_s, bg1, Wg2)
    h2p = _l2_gcn(s1ha, s1hb, h1a, h1b, cnt_h, bh1, Wh2)

    s2e = _sc_segsum(b2t, *eb)
    s2s = _sc_segsum(g2p, *sb)
    s2h = _sc_segsum(h2p, *hb)

    return _final(a2, s2e, cnt_e, b2, s2s, g2p, cnt_s, bg2, s2h, h2p, cnt_h, bh2)
